# Initial kernel scaffold; baseline (speedup 1.0000x reference)
import jax
import jax.numpy as jnp
from jax.experimental import pallas as pl


def kernel(z, edge_index, rel_id, rel):
    raise NotImplementedError("write your pallas kernel here")


# speedup vs baseline: 1.1445x; 1.1445x over previous
import functools

import jax
import jax.numpy as jnp
from jax import lax
from jax.experimental import pallas as pl
from jax.experimental.pallas import tpu as pltpu, tpu_sc as plsc

N_NODES = 10000
N_EDGES = 320000
D = 128
REL_TYPES = 16

_info = plsc.get_sparse_core_info()
NC, NS, L = _info.num_cores, _info.num_subcores, _info.num_lanes
NW = NC * NS
PER_W = N_EDGES // NW
CHUNK = 80
N_CHUNKS = PER_W // CHUNK
GROUPS = CHUNK // L


@functools.partial(
    pl.kernel,
    mesh=plsc.VectorSubcoreMesh(core_axis_name="c", subcore_axis_name="s"),
    out_type=jax.ShapeDtypeStruct((N_EDGES,), jnp.float32),
    scratch_types=[
        pltpu.VMEM((CHUNK,), jnp.int32),
        pltpu.VMEM((CHUNK,), jnp.int32),
        pltpu.VMEM((CHUNK,), jnp.int32),
        pltpu.VMEM((CHUNK, D), jnp.float32),
        pltpu.VMEM((CHUNK, D), jnp.float32),
        pltpu.VMEM((REL_TYPES, D), jnp.float32),
        pltpu.VMEM((CHUNK,), jnp.float32),
        pltpu.SemaphoreType.DMA,
        pltpu.SemaphoreType.DMA,
    ],
    compiler_params=pltpu.CompilerParams(needs_layout_passes=False),
)
def _distmult_sc(z_hbm, src_hbm, dst_hbm, rid_hbm, rel_hbm, out_hbm,
                 srci_v, dsti_v, rid_v, srcr_v, dstr_v, rel_v, out_v,
                 sem_a, sem_b):
    wid = lax.axis_index("s") * NC + lax.axis_index("c")
    base_w = wid * PER_W
    pltpu.sync_copy(rel_hbm, rel_v)
    iota16 = lax.iota(jnp.int32, L)

    def chunk_body(ci, carry):
        base = base_w + ci * CHUNK
        pltpu.sync_copy(src_hbm.at[pl.ds(base, CHUNK)], srci_v)
        pltpu.sync_copy(dst_hbm.at[pl.ds(base, CHUNK)], dsti_v)
        pltpu.sync_copy(rid_hbm.at[pl.ds(base, CHUNK)], rid_v)
        cp_a = pltpu.async_copy(z_hbm.at[srci_v], srcr_v, sem_a)
        cp_b = pltpu.async_copy(z_hbm.at[dsti_v], dstr_v, sem_b)
        cp_a.wait()
        cp_b.wait()

        def group_body(g, carry2):
            lanes = g * L + iota16
            rid = rid_v[pl.ds(g * L, L)]
            acc = jnp.zeros((L,), jnp.float32)
            for j in range(D):
                jv = jnp.full((L,), j, jnp.int32)
                s = plsc.load_gather(srcr_v, [lanes, jv])
                r = plsc.load_gather(rel_v, [rid, jv])
                t = plsc.load_gather(dstr_v, [lanes, jv])
                acc = acc + s * r * t
            sig = 1.0 / (1.0 + jnp.exp(-acc))
            out_v[pl.ds(g * L, L)] = sig
            return carry2

        lax.fori_loop(0, GROUPS, group_body, 0)
        pltpu.sync_copy(out_v, out_hbm.at[pl.ds(base, CHUNK)])
        return carry

    lax.fori_loop(0, N_CHUNKS, chunk_body, 0)


def kernel(z, edge_index, rel_id, rel):
    src = edge_index[0].astype(jnp.int32)
    dst = edge_index[1].astype(jnp.int32)
    rid = rel_id.astype(jnp.int32)
    return _distmult_sc(z, src, dst, rid, rel.astype(jnp.float32))


# speedup vs baseline: 1.3557x; 1.1846x over previous
import functools

import jax
import jax.numpy as jnp
from jax import lax
from jax.experimental import pallas as pl
from jax.experimental.pallas import tpu as pltpu, tpu_sc as plsc

N_NODES = 10000
N_EDGES = 320000
D = 128
REL_TYPES = 16

_info = plsc.get_sparse_core_info()
NC, NS, L = _info.num_cores, _info.num_subcores, _info.num_lanes
NW = NC * NS
PER_W = N_EDGES // NW
CHUNK = 80
N_CHUNKS = PER_W // CHUNK
GROUPS = CHUNK // L


@functools.partial(
    pl.kernel,
    mesh=plsc.VectorSubcoreMesh(core_axis_name="c", subcore_axis_name="s"),
    out_type=jax.ShapeDtypeStruct((N_EDGES,), jnp.float32),
    scratch_types=[
        pltpu.VMEM((N_CHUNKS, CHUNK), jnp.int32),
        pltpu.VMEM((N_CHUNKS, CHUNK), jnp.int32),
        pltpu.VMEM((N_CHUNKS, CHUNK), jnp.int32),
        pltpu.VMEM((CHUNK, D), jnp.float32),
        pltpu.VMEM((CHUNK, D), jnp.float32),
        pltpu.VMEM((CHUNK, D), jnp.float32),
        pltpu.VMEM((CHUNK, D), jnp.float32),
        pltpu.VMEM((REL_TYPES, D), jnp.float32),
        pltpu.VMEM((PER_W,), jnp.float32),
        pltpu.SemaphoreType.DMA,
        pltpu.SemaphoreType.DMA,
        pltpu.SemaphoreType.DMA,
        pltpu.SemaphoreType.DMA,
    ],
    compiler_params=pltpu.CompilerParams(needs_layout_passes=False),
)
def _distmult_sc(z_hbm, src_hbm, dst_hbm, rid_hbm, rel_hbm, out_hbm,
                 srci_v, dsti_v, rid_v, srcr0, srcr1, dstr0, dstr1,
                 rel_v, out_v, sem_s0, sem_s1, sem_d0, sem_d1):
    wid = lax.axis_index("s") * NC + lax.axis_index("c")
    pltpu.sync_copy(rel_hbm, rel_v)
    pltpu.sync_copy(src_hbm.at[wid], srci_v)
    pltpu.sync_copy(dst_hbm.at[wid], dsti_v)
    pltpu.sync_copy(rid_hbm.at[wid], rid_v)
    iota16 = lax.iota(jnp.int32, L)

    srcr = (srcr0, srcr1)
    dstr = (dstr0, dstr1)
    sem_s = (sem_s0, sem_s1)
    sem_d = (sem_d0, sem_d1)

    def issue(ci, b):
        pltpu.async_copy(z_hbm.at[srci_v.at[ci]], srcr[b], sem_s[b])
        pltpu.async_copy(z_hbm.at[dsti_v.at[ci]], dstr[b], sem_d[b])

    def drain(b):
        pltpu.make_async_copy(z_hbm.at[srci_v.at[0]], srcr[b], sem_s[b]).wait()
        pltpu.make_async_copy(z_hbm.at[dsti_v.at[0]], dstr[b], sem_d[b]).wait()

    def compute(ci, b):
        def group_body(g, carry):
            lanes = g * L + iota16
            rid = rid_v[ci, pl.ds(g * L, L)]
            acc = jnp.zeros((L,), jnp.float32)
            for j in range(D):
                jv = jnp.full((L,), j, jnp.int32)
                s = plsc.load_gather(srcr[b], [lanes, jv])
                r = plsc.load_gather(rel_v, [rid, jv])
                t = plsc.load_gather(dstr[b], [lanes, jv])
                acc = acc + s * r * t
            sig = 1.0 / (1.0 + jnp.exp(-acc))
            out_v[pl.ds(ci * CHUNK + g * L, L)] = sig
            return carry

        lax.fori_loop(0, GROUPS, group_body, 0)

    issue(0, 0)

    def body(k, carry):
        ci0 = 2 * k
        ci1 = 2 * k + 1
        issue(ci1, 1)
        drain(0)
        compute(ci0, 0)

        @pl.when(ci0 + 2 < N_CHUNKS)
        def _():
            issue(ci0 + 2, 0)

        drain(1)
        compute(ci1, 1)
        return carry

    lax.fori_loop(0, N_CHUNKS // 2, body, 0)
    drain(0)
    compute(N_CHUNKS - 1, 0)
    pltpu.sync_copy(out_v, out_hbm.at[pl.ds(wid * PER_W, PER_W)])


def kernel(z, edge_index, rel_id, rel):
    src = edge_index[0].astype(jnp.int32).reshape(NW, N_CHUNKS, CHUNK)
    dst = edge_index[1].astype(jnp.int32).reshape(NW, N_CHUNKS, CHUNK)
    rid = rel_id.astype(jnp.int32).reshape(NW, N_CHUNKS, CHUNK)
    return _distmult_sc(z, src, dst, rid, rel.astype(jnp.float32))


# speedup vs baseline: 2.7655x; 2.0399x over previous
import functools

import jax
import jax.numpy as jnp
from jax import lax
from jax.experimental import pallas as pl
from jax.experimental.pallas import tpu as pltpu, tpu_sc as plsc

N_NODES = 10000
N_EDGES = 320000
D = 128
REL_TYPES = 16

_info = plsc.get_sparse_core_info()
NC, NS, L = _info.num_cores, _info.num_subcores, _info.num_lanes
NW = NC * NS
PER_W = N_EDGES // NW
CHUNK = 80
N_CHUNKS = PER_W // CHUNK
GROUPS = CHUNK // L


@functools.partial(
    pl.kernel,
    mesh=plsc.VectorSubcoreMesh(core_axis_name="c", subcore_axis_name="s"),
    out_type=jax.ShapeDtypeStruct((N_EDGES,), jnp.float32),
    scratch_types=[
        pltpu.VMEM((N_CHUNKS, CHUNK), jnp.int32),
        pltpu.VMEM((N_CHUNKS, CHUNK), jnp.int32),
        pltpu.VMEM((N_CHUNKS, CHUNK), jnp.int32),
        pltpu.VMEM((CHUNK, D), jnp.float32),
        pltpu.VMEM((CHUNK, D), jnp.float32),
        pltpu.VMEM((CHUNK, D), jnp.float32),
        pltpu.VMEM((CHUNK, D), jnp.float32),
        pltpu.VMEM((D, REL_TYPES + 1), jnp.float32),
        pltpu.VMEM((PER_W,), jnp.float32),
        pltpu.SemaphoreType.DMA,
        pltpu.SemaphoreType.DMA,
        pltpu.SemaphoreType.DMA,
        pltpu.SemaphoreType.DMA,
    ],
    compiler_params=pltpu.CompilerParams(needs_layout_passes=False),
)
def _distmult_sc(z_hbm, src_hbm, dst_hbm, rid_hbm, rel_hbm, out_hbm,
                 srci_v, dsti_v, rid_v, srcr0, srcr1, dstr0, dstr1,
                 rel_v, out_v, sem_s0, sem_s1, sem_d0, sem_d1):
    wid = lax.axis_index("s") * NC + lax.axis_index("c")
    pltpu.sync_copy(rel_hbm, rel_v)
    pltpu.sync_copy(src_hbm.at[wid], srci_v)
    pltpu.sync_copy(dst_hbm.at[wid], dsti_v)
    pltpu.sync_copy(rid_hbm.at[wid], rid_v)
    iota16 = lax.iota(jnp.int32, L)

    srcr = (srcr0, srcr1)
    dstr = (dstr0, dstr1)
    sem_s = (sem_s0, sem_s1)
    sem_d = (sem_d0, sem_d1)

    def issue(ci, b):
        pltpu.async_copy(z_hbm.at[srci_v.at[ci]], srcr[b], sem_s[b])
        pltpu.async_copy(z_hbm.at[dsti_v.at[ci]], dstr[b], sem_d[b])

    def drain(b):
        pltpu.make_async_copy(z_hbm.at[srci_v.at[0]], srcr[b], sem_s[b]).wait()
        pltpu.make_async_copy(z_hbm.at[dsti_v.at[0]], dstr[b], sem_d[b]).wait()

    def compute(ci, b):
        def group_body(g, carry):
            e0 = g * L
            rid = rid_v[ci, pl.ds(e0, L)]

            def edge_body(k, res):
                kv = jnp.full((L,), 0, jnp.int32) + k
                rid_b = rid.at[kv].get(mode="promise_in_bounds")
                acc = jnp.zeros((L,), jnp.float32)
                for j in range(D // L):
                    s = srcr[b][e0 + k, pl.ds(j * L, L)]
                    t = dstr[b][e0 + k, pl.ds(j * L, L)]
                    r = plsc.load_gather(rel_v, [j * L + iota16, rid_b])
                    acc = acc + s * r * t
                tot = jnp.sum(acc)
                return jnp.where(iota16 == k, tot, res)

            res = lax.fori_loop(0, L, edge_body, jnp.zeros((L,), jnp.float32))
            sig = 1.0 / (1.0 + jnp.exp(-res))
            out_v[pl.ds(ci * CHUNK + e0, L)] = sig
            return carry

        lax.fori_loop(0, GROUPS, group_body, 0)

    issue(0, 0)

    def body(k, carry):
        ci0 = 2 * k
        ci1 = 2 * k + 1
        issue(ci1, 1)
        drain(0)
        compute(ci0, 0)

        @pl.when(ci0 + 2 < N_CHUNKS)
        def _():
            issue(ci0 + 2, 0)

        drain(1)
        compute(ci1, 1)
        return carry

    lax.fori_loop(0, N_CHUNKS // 2, body, 0)
    drain(0)
    compute(N_CHUNKS - 1, 0)
    pltpu.sync_copy(out_v, out_hbm.at[pl.ds(wid * PER_W, PER_W)])


def kernel(z, edge_index, rel_id, rel):
    src = edge_index[0].astype(jnp.int32).reshape(NW, N_CHUNKS, CHUNK)
    dst = edge_index[1].astype(jnp.int32).reshape(NW, N_CHUNKS, CHUNK)
    rid = rel_id.astype(jnp.int32).reshape(NW, N_CHUNKS, CHUNK)
    rel_t = jnp.pad(rel.astype(jnp.float32).T, ((0, 0), (0, 1)))
    return _distmult_sc(z, src, dst, rid, rel_t)


# speedup vs baseline: 2.8721x; 1.0386x over previous
import functools

import jax
import jax.numpy as jnp
from jax import lax
from jax.experimental import pallas as pl
from jax.experimental.pallas import tpu as pltpu, tpu_sc as plsc

N_NODES = 10000
N_EDGES = 320000
D = 128
REL_TYPES = 16

_info = plsc.get_sparse_core_info()
NC, NS, L = _info.num_cores, _info.num_subcores, _info.num_lanes
NW = NC * NS
PER_W = N_EDGES // NW
CHUNK = 80
N_CHUNKS = PER_W // CHUNK
GROUPS = CHUNK // L


@functools.partial(
    pl.kernel,
    mesh=plsc.VectorSubcoreMesh(core_axis_name="c", subcore_axis_name="s"),
    out_type=jax.ShapeDtypeStruct((N_EDGES,), jnp.float32),
    scratch_types=[
        pltpu.VMEM((N_CHUNKS, CHUNK), jnp.int32),
        pltpu.VMEM((N_CHUNKS, CHUNK), jnp.int32),
        pltpu.VMEM((N_CHUNKS, CHUNK), jnp.int32),
        pltpu.VMEM((CHUNK, D), jnp.float32),
        pltpu.VMEM((CHUNK, D), jnp.float32),
        pltpu.VMEM((CHUNK, D), jnp.float32),
        pltpu.VMEM((CHUNK, D), jnp.float32),
        pltpu.VMEM((D, REL_TYPES + 1), jnp.float32),
        pltpu.VMEM((PER_W,), jnp.float32),
        pltpu.SemaphoreType.DMA,
        pltpu.SemaphoreType.DMA,
        pltpu.SemaphoreType.DMA,
        pltpu.SemaphoreType.DMA,
    ],
    compiler_params=pltpu.CompilerParams(needs_layout_passes=False),
)
def _distmult_sc(z_hbm, src_hbm, dst_hbm, rid_hbm, rel_hbm, out_hbm,
                 srci_v, dsti_v, rid_v, srcr0, srcr1, dstr0, dstr1,
                 rel_v, out_v, sem_s0, sem_s1, sem_d0, sem_d1):
    wid = lax.axis_index("s") * NC + lax.axis_index("c")
    pltpu.sync_copy(rel_hbm, rel_v)
    pltpu.sync_copy(src_hbm.at[wid], srci_v)
    pltpu.sync_copy(dst_hbm.at[wid], dsti_v)
    pltpu.sync_copy(rid_hbm.at[wid], rid_v)
    iota16 = lax.iota(jnp.int32, L)

    srcr = (srcr0, srcr1)
    dstr = (dstr0, dstr1)
    sem_s = (sem_s0, sem_s1)
    sem_d = (sem_d0, sem_d1)

    def issue(ci, b):
        pltpu.async_copy(z_hbm.at[srci_v.at[ci]], srcr[b], sem_s[b])
        pltpu.async_copy(z_hbm.at[dsti_v.at[ci]], dstr[b], sem_d[b])

    def drain(b):
        pltpu.make_async_copy(z_hbm.at[srci_v.at[0]], srcr[b], sem_s[b]).wait()
        pltpu.make_async_copy(z_hbm.at[dsti_v.at[0]], dstr[b], sem_d[b]).wait()

    def compute(ci, b):
        def group_body(g, carry):
            e0 = g * L
            rid = rid_v[ci, pl.ds(e0, L)]

            def quad_body(q, res):
                for u in range(4):
                    k = q * 4 + u
                    kv = jnp.zeros((L,), jnp.int32) + k
                    rid_b = rid.at[kv].get(mode="promise_in_bounds")
                    acc0 = jnp.zeros((L,), jnp.float32)
                    acc1 = jnp.zeros((L,), jnp.float32)
                    for j in range(D // L):
                        s = srcr[b][e0 + k, pl.ds(j * L, L)]
                        t = dstr[b][e0 + k, pl.ds(j * L, L)]
                        r = plsc.load_gather(rel_v, [j * L + iota16, rid_b])
                        if j % 2 == 0:
                            acc0 = acc0 + s * r * t
                        else:
                            acc1 = acc1 + s * r * t
                    tot = jnp.sum(acc0 + acc1)
                    res = jnp.where(iota16 == k, tot, res)
                return res

            res = lax.fori_loop(0, L // 4, quad_body,
                                jnp.zeros((L,), jnp.float32))
            sig = 1.0 / (1.0 + jnp.exp(-res))
            out_v[pl.ds(ci * CHUNK + e0, L)] = sig
            return carry

        lax.fori_loop(0, GROUPS, group_body, 0)

    issue(0, 0)

    def body(k, carry):
        ci0 = 2 * k
        ci1 = 2 * k + 1
        issue(ci1, 1)
        drain(0)
        compute(ci0, 0)

        @pl.when(ci0 + 2 < N_CHUNKS)
        def _():
            issue(ci0 + 2, 0)

        drain(1)
        compute(ci1, 1)
        return carry

    lax.fori_loop(0, N_CHUNKS // 2, body, 0)
    drain(0)
    compute(N_CHUNKS - 1, 0)
    pltpu.sync_copy(out_v, out_hbm.at[pl.ds(wid * PER_W, PER_W)])


def kernel(z, edge_index, rel_id, rel):
    src = edge_index[0].astype(jnp.int32).reshape(NW, N_CHUNKS, CHUNK)
    dst = edge_index[1].astype(jnp.int32).reshape(NW, N_CHUNKS, CHUNK)
    rid = rel_id.astype(jnp.int32).reshape(NW, N_CHUNKS, CHUNK)
    rel_t = jnp.pad(rel.astype(jnp.float32).T, ((0, 0), (0, 1)))
    return _distmult_sc(z, src, dst, rid, rel_t)


# speedup vs baseline: 10.0758x; 3.5081x over previous
import functools

import jax
import jax.numpy as jnp
import numpy as np
from jax import lax
from jax.experimental import pallas as pl
from jax.experimental.pallas import tpu as pltpu, tpu_sc as plsc

N_NODES = 10000
N_EDGES = 320000
D = 128
REL_TYPES = 16
RSTR = REL_TYPES + 1

_info = plsc.get_sparse_core_info()
NC, NS, L = _info.num_cores, _info.num_subcores, _info.num_lanes
NW = NC * NS
PER_W = N_EDGES // NW
CHUNK = 80
N_CHUNKS = PER_W // CHUNK
GROUPS = CHUNK // L
ASTR = L + 1


@functools.partial(
    pl.kernel,
    mesh=plsc.VectorSubcoreMesh(core_axis_name="c", subcore_axis_name="s"),
    out_type=jax.ShapeDtypeStruct((N_EDGES,), jnp.float32),
    scratch_types=[
        pltpu.VMEM((PER_W,), jnp.int32),
        pltpu.VMEM((PER_W,), jnp.int32),
        pltpu.VMEM((PER_W,), jnp.int32),
        pltpu.VMEM((CHUNK, D), jnp.bfloat16),
        pltpu.VMEM((CHUNK, D), jnp.bfloat16),
        pltpu.VMEM((CHUNK, D), jnp.bfloat16),
        pltpu.VMEM((CHUNK, D), jnp.bfloat16),
        pltpu.VMEM((D * RSTR,), jnp.float32),
        pltpu.VMEM((L * ASTR,), jnp.float32),
        pltpu.VMEM((PER_W,), jnp.float32),
        pltpu.SemaphoreType.DMA,
        pltpu.SemaphoreType.DMA,
        pltpu.SemaphoreType.DMA,
        pltpu.SemaphoreType.DMA,
    ],
    compiler_params=pltpu.CompilerParams(needs_layout_passes=False,
                                         use_tc_tiling_on_sc=False),
)
def _distmult_sc(z_hbm, src_hbm, dst_hbm, rid_hbm, rel_hbm, out_hbm,
                 srci_v, dsti_v, rid_v, srcr0, srcr1, dstr0, dstr1,
                 rel_v, accs_v, out_v, sem_s0, sem_s1, sem_d0, sem_d1):
    wid = lax.axis_index("s") * NC + lax.axis_index("c")
    base_w = wid * PER_W
    pltpu.sync_copy(rel_hbm, rel_v)
    pltpu.sync_copy(src_hbm.at[pl.ds(base_w, PER_W)], srci_v)
    pltpu.sync_copy(dst_hbm.at[pl.ds(base_w, PER_W)], dsti_v)
    pltpu.sync_copy(rid_hbm.at[pl.ds(base_w, PER_W)], rid_v)
    iota16 = lax.iota(jnp.int32, L)
    iota_astr = iota16 * ASTR
    pv = [[(32 * m + 16 * h + iota16) * RSTR for h in range(2)]
          for m in range(D // 32)]

    srcr = (srcr0, srcr1)
    dstr = (dstr0, dstr1)
    sem_s = (sem_s0, sem_s1)
    sem_d = (sem_d0, sem_d1)

    def issue(ci, b):
        idx_s = srci_v.at[pl.ds(ci * CHUNK, CHUNK)]
        idx_d = dsti_v.at[pl.ds(ci * CHUNK, CHUNK)]
        pltpu.async_copy(z_hbm.at[idx_s], srcr[b], sem_s[b])
        pltpu.async_copy(z_hbm.at[idx_d], dstr[b], sem_d[b])

    def drain(b):
        idx0 = srci_v.at[pl.ds(0, CHUNK)]
        pltpu.make_async_copy(z_hbm.at[idx0], srcr[b], sem_s[b]).wait()
        pltpu.make_async_copy(z_hbm.at[idx0], dstr[b], sem_d[b]).wait()

    def compute(ci, b):
        def group_body(g, carry):
            e0 = g * L
            rid = rid_v[pl.ds(ci * CHUNK + e0, L)]

            def quad_body(q, carry2):
                for u in range(4):
                    k = q * 4 + u
                    kv = jnp.zeros((L,), jnp.int32) + k
                    rid_b = rid.at[kv].get(mode="promise_in_bounds")
                    acc0 = jnp.zeros((L,), jnp.float32)
                    acc1 = jnp.zeros((L,), jnp.float32)
                    for m in range(D // 32):
                        sv = srcr[b][e0 + k, pl.ds(m * 32, 32)]
                        tv = dstr[b][e0 + k, pl.ds(m * 32, 32)]
                        se, so = plsc.unpack(
                            sv, format=plsc.PackFormat.INTERLEAVED)
                        te, to = plsc.unpack(
                            tv, format=plsc.PackFormat.INTERLEAVED)
                        re = plsc.load_gather(rel_v, [pv[m][0] + rid_b])
                        ro = plsc.load_gather(rel_v, [pv[m][1] + rid_b])
                        acc0 = acc0 + (se * te) * re
                        acc1 = acc1 + (so * to) * ro
                    accs_v[pl.ds(k * ASTR, L)] = acc0 + acc1
                return carry2

            lax.fori_loop(0, L // 4, quad_body, 0)
            tot = jnp.zeros((L,), jnp.float32)
            for c in range(L):
                tot = tot + plsc.load_gather(accs_v, [iota_astr + c])
            sig = 1.0 / (1.0 + jnp.exp(-tot))
            out_v[pl.ds(ci * CHUNK + e0, L)] = sig
            return carry

        lax.fori_loop(0, GROUPS, group_body, 0)

    issue(0, 0)

    def body(k, carry):
        ci0 = 2 * k
        ci1 = 2 * k + 1
        issue(ci1, 1)
        drain(0)
        compute(ci0, 0)

        @pl.when(ci0 + 2 < N_CHUNKS)
        def _():
            issue(ci0 + 2, 0)

        drain(1)
        compute(ci1, 1)
        return carry

    lax.fori_loop(0, N_CHUNKS // 2, body, 0)
    drain(0)
    compute(N_CHUNKS - 1, 0)
    pltpu.sync_copy(out_v, out_hbm.at[pl.ds(base_w, PER_W)])


def _rel_perm() -> np.ndarray:
    p = np.arange(D)
    return 32 * (p // 32) + 2 * (p % 16) + ((p % 32) // 16)


def kernel(z, edge_index, rel_id, rel):
    src = edge_index[0].astype(jnp.int32)
    dst = edge_index[1].astype(jnp.int32)
    rid = rel_id.astype(jnp.int32)
    z_bf = z.astype(jnp.bfloat16)
    rel_t = jnp.pad(rel.astype(jnp.float32).T[_rel_perm()],
                    ((0, 0), (0, RSTR - REL_TYPES))).reshape(-1)
    return _distmult_sc(z_bf, src, dst, rid, rel_t)


# speedup vs baseline: 12.6062x; 1.2511x over previous
import functools

import jax
import jax.numpy as jnp
import numpy as np
from jax import lax
from jax.experimental import pallas as pl
from jax.experimental.pallas import tpu as pltpu, tpu_sc as plsc

N_NODES = 10000
N_EDGES = 320000
D = 128
REL_TYPES = 16
RSTR = REL_TYPES + 1

_info = plsc.get_sparse_core_info()
NC, NS, L = _info.num_cores, _info.num_subcores, _info.num_lanes
NW = NC * NS
PER_W = N_EDGES // NW
CHUNK = 80
N_CHUNKS = PER_W // CHUNK
GROUPS = CHUNK // L
ASTR = L + 1


@functools.partial(
    pl.kernel,
    mesh=plsc.VectorSubcoreMesh(core_axis_name="c", subcore_axis_name="s"),
    out_type=jax.ShapeDtypeStruct((N_EDGES,), jnp.float32),
    scratch_types=[
        pltpu.VMEM((PER_W,), jnp.int32),
        pltpu.VMEM((PER_W,), jnp.int32),
        pltpu.VMEM((PER_W,), jnp.int32),
        pltpu.VMEM((CHUNK, D), jnp.bfloat16),
        pltpu.VMEM((CHUNK, D), jnp.bfloat16),
        pltpu.VMEM((CHUNK, D), jnp.bfloat16),
        pltpu.VMEM((CHUNK, D), jnp.bfloat16),
        pltpu.VMEM((D * RSTR,), jnp.float32),
        pltpu.VMEM((L * ASTR,), jnp.float32),
        pltpu.VMEM((PER_W,), jnp.float32),
        pltpu.SemaphoreType.DMA,
        pltpu.SemaphoreType.DMA,
        pltpu.SemaphoreType.DMA,
        pltpu.SemaphoreType.DMA,
    ],
    compiler_params=pltpu.CompilerParams(needs_layout_passes=False,
                                         use_tc_tiling_on_sc=False),
)
def _distmult_sc(z_hbm, src_hbm, dst_hbm, rid_hbm, rel_hbm, out_hbm,
                 srci_v, dsti_v, rid_v, srcr0, srcr1, dstr0, dstr1,
                 rel_v, accs_v, out_v, sem_s0, sem_s1, sem_d0, sem_d1):
    wid = lax.axis_index("s") * NC + lax.axis_index("c")
    base_w = wid * PER_W
    pltpu.sync_copy(rel_hbm, rel_v)
    pltpu.sync_copy(src_hbm.at[pl.ds(base_w, PER_W)], srci_v)
    pltpu.sync_copy(dst_hbm.at[pl.ds(base_w, PER_W)], dsti_v)
    pltpu.sync_copy(rid_hbm.at[pl.ds(base_w, PER_W)], rid_v)
    iota16 = lax.iota(jnp.int32, L)
    iota_astr = iota16 * ASTR
    pv = [[(32 * m + 16 * h + iota16) * RSTR for h in range(2)]
          for m in range(D // 32)]

    srcr = (srcr0, srcr1)
    dstr = (dstr0, dstr1)
    sem_s = (sem_s0, sem_s1)
    sem_d = (sem_d0, sem_d1)

    def issue(ci, b):
        idx_s = srci_v.at[pl.ds(ci * CHUNK, CHUNK)]
        idx_d = dsti_v.at[pl.ds(ci * CHUNK, CHUNK)]
        pltpu.async_copy(z_hbm.at[idx_s], srcr[b], sem_s[b])
        pltpu.async_copy(z_hbm.at[idx_d], dstr[b], sem_d[b])

    def drain(b):
        idx0 = srci_v.at[pl.ds(0, CHUNK)]
        pltpu.make_async_copy(z_hbm.at[idx0], srcr[b], sem_s[b]).wait()
        pltpu.make_async_copy(z_hbm.at[idx0], dstr[b], sem_d[b]).wait()

    def compute(ci, b):
        def group_body(g, carry):
            e0 = g * L
            rid = rid_v[pl.ds(ci * CHUNK + e0, L)]

            @plsc.parallel_loop(0, L, unroll=4)
            def _edge_loop(k):
                kv = jnp.zeros((L,), jnp.int32) + k
                rid_b = rid.at[kv].get(mode="promise_in_bounds")
                acc0 = jnp.zeros((L,), jnp.float32)
                acc1 = jnp.zeros((L,), jnp.float32)
                for m in range(D // 32):
                    sv = srcr[b][e0 + k, pl.ds(m * 32, 32)]
                    tv = dstr[b][e0 + k, pl.ds(m * 32, 32)]
                    se, so = plsc.unpack(
                        sv, format=plsc.PackFormat.INTERLEAVED)
                    te, to = plsc.unpack(
                        tv, format=plsc.PackFormat.INTERLEAVED)
                    re = plsc.load_gather(rel_v, [pv[m][0] + rid_b])
                    ro = plsc.load_gather(rel_v, [pv[m][1] + rid_b])
                    acc0 = acc0 + (se * te) * re
                    acc1 = acc1 + (so * to) * ro
                accs_v[pl.ds(k * ASTR, L)] = acc0 + acc1

            parts = []
            for p4 in range(4):
                t = plsc.load_gather(accs_v, [iota_astr + 4 * p4])
                for c in range(1, 4):
                    t = t + plsc.load_gather(accs_v, [iota_astr + 4 * p4 + c])
                parts.append(t)
            tot = (parts[0] + parts[1]) + (parts[2] + parts[3])
            sig = 1.0 / (1.0 + jnp.exp(-tot))
            out_v[pl.ds(ci * CHUNK + e0, L)] = sig
            return carry

        lax.fori_loop(0, GROUPS, group_body, 0)

    issue(0, 0)

    def body(k, carry):
        ci0 = 2 * k
        ci1 = 2 * k + 1
        issue(ci1, 1)
        drain(0)
        compute(ci0, 0)

        @pl.when(ci0 + 2 < N_CHUNKS)
        def _():
            issue(ci0 + 2, 0)

        drain(1)
        compute(ci1, 1)
        return carry

    lax.fori_loop(0, N_CHUNKS // 2, body, 0)
    drain(0)
    compute(N_CHUNKS - 1, 0)
    pltpu.sync_copy(out_v, out_hbm.at[pl.ds(base_w, PER_W)])


def _rel_perm() -> np.ndarray:
    p = np.arange(D)
    return 32 * (p // 32) + 2 * (p % 16) + ((p % 32) // 16)


def kernel(z, edge_index, rel_id, rel):
    src = edge_index[0].astype(jnp.int32)
    dst = edge_index[1].astype(jnp.int32)
    rid = rel_id.astype(jnp.int32)
    z_bf = z.astype(jnp.bfloat16)
    rel_t = jnp.pad(rel.astype(jnp.float32).T[_rel_perm()],
                    ((0, 0), (0, RSTR - REL_TYPES))).reshape(-1)
    return _distmult_sc(z_bf, src, dst, rid, rel_t)


# speedup vs baseline: 13.5283x; 1.0731x over previous
import functools

import jax
import jax.numpy as jnp
import numpy as np
from jax import lax
from jax.experimental import pallas as pl
from jax.experimental.pallas import tpu as pltpu, tpu_sc as plsc

N_NODES = 10000
N_EDGES = 320000
D = 128
REL_TYPES = 16
RSTR = REL_TYPES + 1

_info = plsc.get_sparse_core_info()
NC, NS, L = _info.num_cores, _info.num_subcores, _info.num_lanes
NW = NC * NS
PER_W = N_EDGES // NW
CHUNK = 80
N_CHUNKS = PER_W // CHUNK
GROUPS = CHUNK // L
ASTR = L + 1


@functools.partial(
    pl.kernel,
    mesh=plsc.VectorSubcoreMesh(core_axis_name="c", subcore_axis_name="s"),
    out_type=jax.ShapeDtypeStruct((N_EDGES,), jnp.float32),
    scratch_types=[
        pltpu.VMEM((PER_W,), jnp.int32),
        pltpu.VMEM((PER_W,), jnp.int32),
        pltpu.VMEM((PER_W,), jnp.int32),
        pltpu.VMEM((CHUNK, D), jnp.bfloat16),
        pltpu.VMEM((CHUNK, D), jnp.bfloat16),
        pltpu.VMEM((CHUNK, D), jnp.bfloat16),
        pltpu.VMEM((CHUNK, D), jnp.bfloat16),
        pltpu.VMEM((D * RSTR,), jnp.float32),
        pltpu.VMEM((CHUNK * ASTR,), jnp.float32),
        pltpu.VMEM((PER_W,), jnp.float32),
        pltpu.SemaphoreType.DMA,
        pltpu.SemaphoreType.DMA,
        pltpu.SemaphoreType.DMA,
        pltpu.SemaphoreType.DMA,
    ],
    compiler_params=pltpu.CompilerParams(needs_layout_passes=False,
                                         use_tc_tiling_on_sc=False),
)
def _distmult_sc(z_hbm, src_hbm, dst_hbm, rid_hbm, rel_hbm, out_hbm,
                 srci_v, dsti_v, rid_v, srcr0, srcr1, dstr0, dstr1,
                 rel_v, accs_v, out_v, sem_s0, sem_s1, sem_d0, sem_d1):
    wid = lax.axis_index("s") * NC + lax.axis_index("c")
    base_w = wid * PER_W
    pltpu.sync_copy(rel_hbm, rel_v)
    pltpu.sync_copy(src_hbm.at[pl.ds(base_w, PER_W)], srci_v)
    pltpu.sync_copy(dst_hbm.at[pl.ds(base_w, PER_W)], dsti_v)
    pltpu.sync_copy(rid_hbm.at[pl.ds(base_w, PER_W)], rid_v)
    iota16 = lax.iota(jnp.int32, L)
    iota_astr = iota16 * ASTR
    pv = [[(32 * m + 16 * h + iota16) * RSTR for h in range(2)]
          for m in range(D // 32)]

    srcr = (srcr0, srcr1)
    dstr = (dstr0, dstr1)
    sem_s = (sem_s0, sem_s1)
    sem_d = (sem_d0, sem_d1)

    def issue(ci, b):
        idx_s = srci_v.at[pl.ds(ci * CHUNK, CHUNK)]
        idx_d = dsti_v.at[pl.ds(ci * CHUNK, CHUNK)]
        pltpu.async_copy(z_hbm.at[idx_s], srcr[b], sem_s[b])
        pltpu.async_copy(z_hbm.at[idx_d], dstr[b], sem_d[b])

    def drain(b):
        idx0 = srci_v.at[pl.ds(0, CHUNK)]
        pltpu.make_async_copy(z_hbm.at[idx0], srcr[b], sem_s[b]).wait()
        pltpu.make_async_copy(z_hbm.at[idx0], dstr[b], sem_d[b]).wait()

    def compute(ci, b):
        @plsc.parallel_loop(0, CHUNK, unroll=4)
        def _edge_loop(k):
            g16 = k & ~(L - 1)
            rid_g = rid_v[pl.ds(ci * CHUNK + g16, L)]
            kv = jnp.zeros((L,), jnp.int32) + (k & (L - 1))
            rid_b = rid_g.at[kv].get(mode="promise_in_bounds")
            acc0 = jnp.zeros((L,), jnp.float32)
            acc1 = jnp.zeros((L,), jnp.float32)
            for m in range(D // 32):
                sv = srcr[b][k, pl.ds(m * 32, 32)]
                tv = dstr[b][k, pl.ds(m * 32, 32)]
                se, so = plsc.unpack(sv, format=plsc.PackFormat.INTERLEAVED)
                te, to = plsc.unpack(tv, format=plsc.PackFormat.INTERLEAVED)
                re = plsc.load_gather(rel_v, [pv[m][0] + rid_b])
                ro = plsc.load_gather(rel_v, [pv[m][1] + rid_b])
                acc0 = acc0 + (se * te) * re
                acc1 = acc1 + (so * to) * ro
            accs_v[pl.ds(k * ASTR, L)] = acc0 + acc1

        @plsc.parallel_loop(0, GROUPS, unroll=1)
        def _reduce_loop(g):
            base = g * (L * ASTR)
            parts = []
            for p4 in range(4):
                t = plsc.load_gather(accs_v, [base + iota_astr + 4 * p4])
                for c in range(1, 4):
                    t = t + plsc.load_gather(
                        accs_v, [base + iota_astr + 4 * p4 + c])
                parts.append(t)
            tot = (parts[0] + parts[1]) + (parts[2] + parts[3])
            sig = 1.0 / (1.0 + jnp.exp(-tot))
            out_v[pl.ds(ci * CHUNK + g * L, L)] = sig

    issue(0, 0)

    def body(k, carry):
        ci0 = 2 * k
        ci1 = 2 * k + 1
        issue(ci1, 1)
        drain(0)
        compute(ci0, 0)

        @pl.when(ci0 + 2 < N_CHUNKS)
        def _():
            issue(ci0 + 2, 0)

        drain(1)
        compute(ci1, 1)
        return carry

    lax.fori_loop(0, N_CHUNKS // 2, body, 0)
    drain(0)
    compute(N_CHUNKS - 1, 0)
    pltpu.sync_copy(out_v, out_hbm.at[pl.ds(base_w, PER_W)])


def _rel_perm() -> np.ndarray:
    p = np.arange(D)
    return 32 * (p // 32) + 2 * (p % 16) + ((p % 32) // 16)


def kernel(z, edge_index, rel_id, rel):
    src = edge_index[0].astype(jnp.int32)
    dst = edge_index[1].astype(jnp.int32)
    rid = rel_id.astype(jnp.int32)
    z_bf = z.astype(jnp.bfloat16)
    rel_t = jnp.pad(rel.astype(jnp.float32).T[_rel_perm()],
                    ((0, 0), (0, RSTR - REL_TYPES))).reshape(-1)
    return _distmult_sc(z_bf, src, dst, rid, rel_t)


# speedup vs baseline: 13.8989x; 1.0274x over previous
import functools

import jax
import jax.numpy as jnp
import numpy as np
from jax import lax
from jax.experimental import pallas as pl
from jax.experimental.pallas import tpu as pltpu, tpu_sc as plsc

N_NODES = 10000
N_EDGES = 320000
D = 128
REL_TYPES = 16
RSTR = REL_TYPES + 1

_info = plsc.get_sparse_core_info()
NC, NS, L = _info.num_cores, _info.num_subcores, _info.num_lanes
NW = NC * NS
PER_W = N_EDGES // NW
CHUNK = 80
N_CHUNKS = PER_W // CHUNK
GROUPS = CHUNK // L
ASTR = L + 1


@functools.partial(
    pl.kernel,
    mesh=plsc.VectorSubcoreMesh(core_axis_name="c", subcore_axis_name="s"),
    out_type=jax.ShapeDtypeStruct((N_EDGES,), jnp.float32),
    scratch_types=[
        pltpu.VMEM((PER_W,), jnp.int32),
        pltpu.VMEM((PER_W,), jnp.int32),
        pltpu.VMEM((PER_W,), jnp.int32),
        pltpu.VMEM((CHUNK, D), jnp.bfloat16),
        pltpu.VMEM((CHUNK, D), jnp.bfloat16),
        pltpu.VMEM((CHUNK, D), jnp.bfloat16),
        pltpu.VMEM((CHUNK, D), jnp.bfloat16),
        pltpu.VMEM((D * RSTR,), jnp.float32),
        pltpu.VMEM((CHUNK * ASTR,), jnp.float32),
        pltpu.VMEM((PER_W,), jnp.float32),
        pltpu.SemaphoreType.DMA,
        pltpu.SemaphoreType.DMA,
        pltpu.SemaphoreType.DMA,
        pltpu.SemaphoreType.DMA,
    ],
    compiler_params=pltpu.CompilerParams(needs_layout_passes=False,
                                         use_tc_tiling_on_sc=False),
)
def _distmult_sc(z_hbm, src_hbm, dst_hbm, rid_hbm, rel_hbm, out_hbm,
                 srci_v, dsti_v, rid_v, srcr0, srcr1, dstr0, dstr1,
                 rel_v, accs_v, out_v, sem_s0, sem_s1, sem_d0, sem_d1):
    wid = lax.axis_index("s") * NC + lax.axis_index("c")
    base_w = wid * PER_W
    pltpu.sync_copy(rel_hbm, rel_v)
    pltpu.sync_copy(src_hbm.at[pl.ds(base_w, PER_W)], srci_v)
    pltpu.sync_copy(dst_hbm.at[pl.ds(base_w, PER_W)], dsti_v)
    pltpu.sync_copy(rid_hbm.at[pl.ds(base_w, PER_W)], rid_v)
    iota16 = lax.iota(jnp.int32, L)
    iota_astr = iota16 * ASTR
    pv = [[(32 * m + 16 * h + iota16) * RSTR for h in range(2)]
          for m in range(D // 32)]

    srcr = (srcr0, srcr1)
    dstr = (dstr0, dstr1)
    sem_s = (sem_s0, sem_s1)
    sem_d = (sem_d0, sem_d1)

    def issue(ci, b):
        idx_s = srci_v.at[pl.ds(ci * CHUNK, CHUNK)]
        idx_d = dsti_v.at[pl.ds(ci * CHUNK, CHUNK)]
        pltpu.async_copy(z_hbm.at[idx_s], srcr[b], sem_s[b])
        pltpu.async_copy(z_hbm.at[idx_d], dstr[b], sem_d[b])

    def drain(b):
        idx0 = srci_v.at[pl.ds(0, CHUNK)]
        pltpu.make_async_copy(z_hbm.at[idx0], srcr[b], sem_s[b]).wait()
        pltpu.make_async_copy(z_hbm.at[idx0], dstr[b], sem_d[b]).wait()

    def compute(ci, b):
        @plsc.parallel_loop(0, CHUNK, unroll=8)
        def _edge_loop(k):
            g16 = k & ~(L - 1)
            rid_g = rid_v[pl.ds(ci * CHUNK + g16, L)]
            kv = jnp.zeros((L,), jnp.int32) + (k & (L - 1))
            rid_b = rid_g.at[kv].get(mode="promise_in_bounds")
            acc0 = jnp.zeros((L,), jnp.float32)
            acc1 = jnp.zeros((L,), jnp.float32)
            for m in range(D // 32):
                sv = srcr[b][k, pl.ds(m * 32, 32)]
                tv = dstr[b][k, pl.ds(m * 32, 32)]
                se, so = plsc.unpack(sv, format=plsc.PackFormat.INTERLEAVED)
                te, to = plsc.unpack(tv, format=plsc.PackFormat.INTERLEAVED)
                re = plsc.load_gather(rel_v, [pv[m][0] + rid_b])
                ro = plsc.load_gather(rel_v, [pv[m][1] + rid_b])
                acc0 = acc0 + (se * te) * re
                acc1 = acc1 + (so * to) * ro
            accs_v[pl.ds(k * ASTR, L)] = acc0 + acc1

        @plsc.parallel_loop(0, GROUPS, unroll=1)
        def _reduce_loop(g):
            base = g * (L * ASTR)
            parts = []
            for p4 in range(4):
                t = plsc.load_gather(accs_v, [base + iota_astr + 4 * p4])
                for c in range(1, 4):
                    t = t + plsc.load_gather(
                        accs_v, [base + iota_astr + 4 * p4 + c])
                parts.append(t)
            tot = (parts[0] + parts[1]) + (parts[2] + parts[3])
            sig = 1.0 / (1.0 + jnp.exp(-tot))
            out_v[pl.ds(ci * CHUNK + g * L, L)] = sig

    issue(0, 0)

    def body(k, carry):
        ci0 = 2 * k
        ci1 = 2 * k + 1
        issue(ci1, 1)
        drain(0)
        compute(ci0, 0)

        @pl.when(ci0 + 2 < N_CHUNKS)
        def _():
            issue(ci0 + 2, 0)

        drain(1)
        compute(ci1, 1)
        return carry

    lax.fori_loop(0, N_CHUNKS // 2, body, 0)
    drain(0)
    compute(N_CHUNKS - 1, 0)
    pltpu.sync_copy(out_v, out_hbm.at[pl.ds(base_w, PER_W)])


def _rel_perm() -> np.ndarray:
    p = np.arange(D)
    return 32 * (p // 32) + 2 * (p % 16) + ((p % 32) // 16)


def kernel(z, edge_index, rel_id, rel):
    src = edge_index[0].astype(jnp.int32)
    dst = edge_index[1].astype(jnp.int32)
    rid = rel_id.astype(jnp.int32)
    z_bf = z.astype(jnp.bfloat16)
    rel_t = jnp.pad(rel.astype(jnp.float32).T[_rel_perm()],
                    ((0, 0), (0, RSTR - REL_TYPES))).reshape(-1)
    return _distmult_sc(z_bf, src, dst, rid, rel_t)


# speedup vs baseline: 14.7054x; 1.0580x over previous
import functools

import jax
import jax.numpy as jnp
import numpy as np
from jax import lax
from jax.experimental import pallas as pl
from jax.experimental.pallas import tpu as pltpu, tpu_sc as plsc

N_NODES = 10000
N_EDGES = 320000
D = 128
REL_TYPES = 16
RSTR = REL_TYPES + 1

_info = plsc.get_sparse_core_info()
NC, NS, L = _info.num_cores, _info.num_subcores, _info.num_lanes
NW = NC * NS
PER_W = N_EDGES // NW
CHUNK = 80
N_CHUNKS = PER_W // CHUNK
GROUPS = CHUNK // L
ASTR = L + 1


@functools.partial(
    pl.kernel,
    mesh=plsc.VectorSubcoreMesh(core_axis_name="c", subcore_axis_name="s"),
    out_type=jax.ShapeDtypeStruct((N_EDGES,), jnp.float32),
    scratch_types=[
        pltpu.VMEM((PER_W,), jnp.int32),
        pltpu.VMEM((PER_W,), jnp.int32),
        pltpu.VMEM((PER_W,), jnp.int32),
        pltpu.VMEM((CHUNK, D), jnp.bfloat16),
        pltpu.VMEM((CHUNK, D), jnp.bfloat16),
        pltpu.VMEM((CHUNK, D), jnp.bfloat16),
        pltpu.VMEM((CHUNK, D), jnp.bfloat16),
        pltpu.VMEM((D // 2 * RSTR,), jnp.float32),
        pltpu.VMEM((CHUNK * ASTR,), jnp.float32),
        pltpu.VMEM((PER_W,), jnp.float32),
        pltpu.SemaphoreType.DMA,
        pltpu.SemaphoreType.DMA,
        pltpu.SemaphoreType.DMA,
        pltpu.SemaphoreType.DMA,
    ],
    compiler_params=pltpu.CompilerParams(needs_layout_passes=False,
                                         use_tc_tiling_on_sc=False),
)
def _distmult_sc(z_hbm, src_hbm, dst_hbm, rid_hbm, rel_hbm, out_hbm,
                 srci_v, dsti_v, rid_v, srcr0, srcr1, dstr0, dstr1,
                 rel_v, accs_v, out_v, sem_s0, sem_s1, sem_d0, sem_d1):
    wid = lax.axis_index("s") * NC + lax.axis_index("c")
    base_w = wid * PER_W
    pltpu.sync_copy(rel_hbm, rel_v)
    pltpu.sync_copy(src_hbm.at[pl.ds(base_w, PER_W)], srci_v)
    pltpu.sync_copy(dst_hbm.at[pl.ds(base_w, PER_W)], dsti_v)
    pltpu.sync_copy(rid_hbm.at[pl.ds(base_w, PER_W)], rid_v)
    iota16 = lax.iota(jnp.int32, L)
    iota_astr = iota16 * ASTR
    pv = [(16 * m + iota16) * RSTR for m in range(D // 32)]

    srcr = (srcr0, srcr1)
    dstr = (dstr0, dstr1)
    sem_s = (sem_s0, sem_s1)
    sem_d = (sem_d0, sem_d1)

    def issue(ci, b):
        idx_s = srci_v.at[pl.ds(ci * CHUNK, CHUNK)]
        idx_d = dsti_v.at[pl.ds(ci * CHUNK, CHUNK)]
        pltpu.async_copy(z_hbm.at[idx_s], srcr[b], sem_s[b])
        pltpu.async_copy(z_hbm.at[idx_d], dstr[b], sem_d[b])

    def drain(b):
        idx0 = srci_v.at[pl.ds(0, CHUNK)]
        pltpu.make_async_copy(z_hbm.at[idx0], srcr[b], sem_s[b]).wait()
        pltpu.make_async_copy(z_hbm.at[idx0], dstr[b], sem_d[b]).wait()

    def compute(ci, b):
        @plsc.parallel_loop(0, CHUNK, unroll=8)
        def _edge_loop(k):
            g16 = k & ~(L - 1)
            rid_g = rid_v[pl.ds(ci * CHUNK + g16, L)]
            kv = jnp.zeros((L,), jnp.int32) + (k & (L - 1))
            rid_b = rid_g.at[kv].get(mode="promise_in_bounds")
            acc0 = jnp.zeros((L,), jnp.float32)
            acc1 = jnp.zeros((L,), jnp.float32)
            for m in range(D // 32):
                sv = srcr[b][k, pl.ds(m * 32, 32)]
                tv = dstr[b][k, pl.ds(m * 32, 32)]
                pe, po = plsc.unpack(sv * tv,
                                     format=plsc.PackFormat.INTERLEAVED)
                rp = plsc.load_gather(rel_v, [pv[m] + rid_b])
                re, ro = plsc.unpack(plsc.bitcast(rp, jnp.bfloat16),
                                     format=plsc.PackFormat.INTERLEAVED)
                acc0 = acc0 + pe * re
                acc1 = acc1 + po * ro
            accs_v[pl.ds(k * ASTR, L)] = acc0 + acc1

        @plsc.parallel_loop(0, GROUPS, unroll=1)
        def _reduce_loop(g):
            base = g * (L * ASTR)
            parts = []
            for p4 in range(4):
                t = plsc.load_gather(accs_v, [base + iota_astr + 4 * p4])
                for c in range(1, 4):
                    t = t + plsc.load_gather(
                        accs_v, [base + iota_astr + 4 * p4 + c])
                parts.append(t)
            tot = (parts[0] + parts[1]) + (parts[2] + parts[3])
            sig = 1.0 / (1.0 + jnp.exp(-tot))
            out_v[pl.ds(ci * CHUNK + g * L, L)] = sig

    issue(0, 0)

    def body(k, carry):
        ci0 = 2 * k
        ci1 = 2 * k + 1
        issue(ci1, 1)
        drain(0)
        compute(ci0, 0)

        @pl.when(ci0 + 2 < N_CHUNKS)
        def _():
            issue(ci0 + 2, 0)

        drain(1)
        compute(ci1, 1)
        return carry

    lax.fori_loop(0, N_CHUNKS // 2, body, 0)
    drain(0)
    compute(N_CHUNKS - 1, 0)
    pltpu.sync_copy(out_v, out_hbm.at[pl.ds(base_w, PER_W)])


def kernel(z, edge_index, rel_id, rel):
    src = edge_index[0].astype(jnp.int32)
    dst = edge_index[1].astype(jnp.int32)
    rid = rel_id.astype(jnp.int32)
    z_bf = z.astype(jnp.bfloat16)
    q = np.arange(D // 2)
    f_even = 32 * (q // 16) + 2 * (q % 16)
    rt = rel.astype(jnp.float32).T
    e16 = jax.lax.bitcast_convert_type(
        rt[f_even].astype(jnp.bfloat16), jnp.uint16).astype(jnp.uint32)
    o16 = jax.lax.bitcast_convert_type(
        rt[f_even + 1].astype(jnp.bfloat16), jnp.uint16).astype(jnp.uint32)
    packed = jax.lax.bitcast_convert_type(e16 | (o16 << 16), jnp.float32)
    rel_t = jnp.pad(packed, ((0, 0), (0, RSTR - REL_TYPES))).reshape(-1)
    return _distmult_sc(z_bf, src, dst, rid, rel_t)


# speedup vs baseline: 15.1797x; 1.0323x over previous
import functools

import jax
import jax.numpy as jnp
import numpy as np
from jax import lax
from jax.experimental import pallas as pl
from jax.experimental.pallas import tpu as pltpu, tpu_sc as plsc

N_NODES = 10000
N_EDGES = 320000
D = 128
REL_TYPES = 16
RSTR = REL_TYPES + 1

_info = plsc.get_sparse_core_info()
NC, NS, L = _info.num_cores, _info.num_subcores, _info.num_lanes
NW = NC * NS
PER_W = N_EDGES // NW
CHUNK = 80
N_CHUNKS = PER_W // CHUNK
GROUPS = CHUNK // L
ASTR = L + 1


@functools.partial(
    pl.kernel,
    mesh=plsc.VectorSubcoreMesh(core_axis_name="c", subcore_axis_name="s"),
    out_type=jax.ShapeDtypeStruct((N_EDGES,), jnp.float32),
    scratch_types=[
        pltpu.VMEM((PER_W,), jnp.int32),
        pltpu.VMEM((PER_W,), jnp.int32),
        pltpu.VMEM((PER_W,), jnp.int32),
        pltpu.VMEM((CHUNK, D), jnp.bfloat16),
        pltpu.VMEM((CHUNK, D), jnp.bfloat16),
        pltpu.VMEM((CHUNK, D), jnp.bfloat16),
        pltpu.VMEM((CHUNK, D), jnp.bfloat16),
        pltpu.VMEM((D // 2 * RSTR,), jnp.float32),
        pltpu.VMEM((CHUNK * ASTR,), jnp.float32),
        pltpu.VMEM((PER_W,), jnp.float32),
        pltpu.VMEM_SHARED((N_NODES, D), jnp.bfloat16),
        pltpu.SemaphoreType.DMA,
        pltpu.SemaphoreType.DMA,
        pltpu.SemaphoreType.DMA,
        pltpu.SemaphoreType.DMA,
    ],
    compiler_params=pltpu.CompilerParams(needs_layout_passes=False,
                                         use_tc_tiling_on_sc=False),
)
def _distmult_sc(z_hbm, src_hbm, dst_hbm, rid_hbm, rel_hbm, out_hbm,
                 srci_v, dsti_v, rid_v, srcr0, srcr1, dstr0, dstr1,
                 rel_v, accs_v, out_v, z_sh, sem_s0, sem_s1, sem_d0, sem_d1):
    wid = lax.axis_index("s") * NC + lax.axis_index("c")
    base_w = wid * PER_W
    sid = lax.axis_index("s")
    zrows = N_NODES // NS
    pltpu.sync_copy(z_hbm.at[pl.ds(sid * zrows, zrows)],
                    z_sh.at[pl.ds(sid * zrows, zrows)])
    pltpu.sync_copy(rel_hbm, rel_v)
    pltpu.sync_copy(src_hbm.at[pl.ds(base_w, PER_W)], srci_v)
    pltpu.sync_copy(dst_hbm.at[pl.ds(base_w, PER_W)], dsti_v)
    pltpu.sync_copy(rid_hbm.at[pl.ds(base_w, PER_W)], rid_v)
    plsc.subcore_barrier()
    iota16 = lax.iota(jnp.int32, L)
    iota_astr = iota16 * ASTR
    pv = [(16 * m + iota16) * RSTR for m in range(D // 32)]

    srcr = (srcr0, srcr1)
    dstr = (dstr0, dstr1)
    sem_s = (sem_s0, sem_s1)
    sem_d = (sem_d0, sem_d1)

    def issue(ci, b):
        idx_s = srci_v.at[pl.ds(ci * CHUNK, CHUNK)]
        idx_d = dsti_v.at[pl.ds(ci * CHUNK, CHUNK)]
        pltpu.async_copy(z_sh.at[idx_s], srcr[b], sem_s[b])
        pltpu.async_copy(z_sh.at[idx_d], dstr[b], sem_d[b])

    def drain(b):
        idx0 = srci_v.at[pl.ds(0, CHUNK)]
        pltpu.make_async_copy(z_sh.at[idx0], srcr[b], sem_s[b]).wait()
        pltpu.make_async_copy(z_sh.at[idx0], dstr[b], sem_d[b]).wait()

    def compute(ci, b):
        @plsc.parallel_loop(0, CHUNK, unroll=8)
        def _edge_loop(k):
            g16 = k & ~(L - 1)
            rid_g = rid_v[pl.ds(ci * CHUNK + g16, L)]
            kv = jnp.zeros((L,), jnp.int32) + (k & (L - 1))
            rid_b = rid_g.at[kv].get(mode="promise_in_bounds")
            acc0 = jnp.zeros((L,), jnp.float32)
            acc1 = jnp.zeros((L,), jnp.float32)
            for m in range(D // 32):
                sv = srcr[b][k, pl.ds(m * 32, 32)]
                tv = dstr[b][k, pl.ds(m * 32, 32)]
                pe, po = plsc.unpack(sv * tv,
                                     format=plsc.PackFormat.INTERLEAVED)
                rp = plsc.load_gather(rel_v, [pv[m] + rid_b])
                re, ro = plsc.unpack(plsc.bitcast(rp, jnp.bfloat16),
                                     format=plsc.PackFormat.INTERLEAVED)
                acc0 = acc0 + pe * re
                acc1 = acc1 + po * ro
            accs_v[pl.ds(k * ASTR, L)] = acc0 + acc1

        @plsc.parallel_loop(0, GROUPS, unroll=1)
        def _reduce_loop(g):
            base = g * (L * ASTR)
            parts = []
            for p4 in range(4):
                t = plsc.load_gather(accs_v, [base + iota_astr + 4 * p4])
                for c in range(1, 4):
                    t = t + plsc.load_gather(
                        accs_v, [base + iota_astr + 4 * p4 + c])
                parts.append(t)
            tot = (parts[0] + parts[1]) + (parts[2] + parts[3])
            sig = 1.0 / (1.0 + jnp.exp(-tot))
            out_v[pl.ds(ci * CHUNK + g * L, L)] = sig

    issue(0, 0)

    def body(k, carry):
        ci0 = 2 * k
        ci1 = 2 * k + 1
        issue(ci1, 1)
        drain(0)
        compute(ci0, 0)

        @pl.when(ci0 + 2 < N_CHUNKS)
        def _():
            issue(ci0 + 2, 0)

        drain(1)
        compute(ci1, 1)
        return carry

    lax.fori_loop(0, N_CHUNKS // 2, body, 0)
    drain(0)
    compute(N_CHUNKS - 1, 0)
    pltpu.sync_copy(out_v, out_hbm.at[pl.ds(base_w, PER_W)])


def kernel(z, edge_index, rel_id, rel):
    src = edge_index[0].astype(jnp.int32)
    dst = edge_index[1].astype(jnp.int32)
    rid = rel_id.astype(jnp.int32)
    z_bf = z.astype(jnp.bfloat16)
    q = np.arange(D // 2)
    f_even = 32 * (q // 16) + 2 * (q % 16)
    rt = rel.astype(jnp.float32).T
    e16 = jax.lax.bitcast_convert_type(
        rt[f_even].astype(jnp.bfloat16), jnp.uint16).astype(jnp.uint32)
    o16 = jax.lax.bitcast_convert_type(
        rt[f_even + 1].astype(jnp.bfloat16), jnp.uint16).astype(jnp.uint32)
    packed = jax.lax.bitcast_convert_type(e16 | (o16 << 16), jnp.float32)
    rel_t = jnp.pad(packed, ((0, 0), (0, RSTR - REL_TYPES))).reshape(-1)
    return _distmult_sc(z_bf, src, dst, rid, rel_t)


# speedup vs baseline: 15.3852x; 1.0135x over previous
import functools

import jax
import jax.numpy as jnp
import numpy as np
from jax import lax
from jax.experimental import pallas as pl
from jax.experimental.pallas import tpu as pltpu, tpu_sc as plsc

N_NODES = 10000
N_EDGES = 320000
D = 128
REL_TYPES = 16
RSTR = REL_TYPES + 1

_info = plsc.get_sparse_core_info()
NC, NS, L = _info.num_cores, _info.num_subcores, _info.num_lanes
NW = NC * NS
PER_W = N_EDGES // NW
CHUNK = 80
N_CHUNKS = PER_W // CHUNK
GROUPS = CHUNK // L
ASTR = L + 1


@functools.partial(
    pl.kernel,
    mesh=plsc.VectorSubcoreMesh(core_axis_name="c", subcore_axis_name="s"),
    out_type=jax.ShapeDtypeStruct((N_EDGES,), jnp.float32),
    scratch_types=[
        pltpu.VMEM((PER_W,), jnp.int32),
        pltpu.VMEM((PER_W,), jnp.int32),
        pltpu.VMEM((PER_W,), jnp.int32),
        pltpu.VMEM((CHUNK, D), jnp.bfloat16),
        pltpu.VMEM((CHUNK, D), jnp.bfloat16),
        pltpu.VMEM((CHUNK, D), jnp.bfloat16),
        pltpu.VMEM((CHUNK, D), jnp.bfloat16),
        pltpu.VMEM((D // 2 * RSTR,), jnp.float32),
        pltpu.VMEM((CHUNK * ASTR,), jnp.float32),
        pltpu.VMEM((PER_W,), jnp.float32),
        pltpu.VMEM_SHARED((N_NODES, D), jnp.bfloat16),
        pltpu.SemaphoreType.DMA,
        pltpu.SemaphoreType.DMA,
        pltpu.SemaphoreType.DMA,
        pltpu.SemaphoreType.DMA,
    ],
    compiler_params=pltpu.CompilerParams(needs_layout_passes=False,
                                         use_tc_tiling_on_sc=False),
)
def _distmult_sc(z_hbm, src_hbm, dst_hbm, rid_hbm, rel_hbm, out_hbm,
                 srci_v, dsti_v, rid_v, srcr0, srcr1, dstr0, dstr1,
                 rel_v, accs_v, out_v, z_sh, sem_s0, sem_s1, sem_d0, sem_d1):
    wid = lax.axis_index("s") * NC + lax.axis_index("c")
    base_w = wid * PER_W
    sid = lax.axis_index("s")
    zrows = N_NODES // NS
    pltpu.sync_copy(z_hbm.at[pl.ds(sid * zrows, zrows)],
                    z_sh.at[pl.ds(sid * zrows, zrows)])
    pltpu.sync_copy(rel_hbm, rel_v)
    pltpu.sync_copy(src_hbm.at[pl.ds(base_w, PER_W)], srci_v)
    pltpu.sync_copy(dst_hbm.at[pl.ds(base_w, PER_W)], dsti_v)
    pltpu.sync_copy(rid_hbm.at[pl.ds(base_w, PER_W)], rid_v)
    plsc.subcore_barrier()
    iota16 = lax.iota(jnp.int32, L)
    iota_astr = iota16 * ASTR
    pv = [(16 * m + iota16) * RSTR for m in range(D // 32)]

    srcr = (srcr0, srcr1)
    dstr = (dstr0, dstr1)
    sem_s = (sem_s0, sem_s1)
    sem_d = (sem_d0, sem_d1)

    def issue(ci, b):
        idx_s = srci_v.at[pl.ds(ci * CHUNK, CHUNK)]
        idx_d = dsti_v.at[pl.ds(ci * CHUNK, CHUNK)]
        pltpu.async_copy(z_sh.at[idx_s], srcr[b], sem_s[b])
        pltpu.async_copy(z_sh.at[idx_d], dstr[b], sem_d[b])

    def drain(b):
        idx0 = srci_v.at[pl.ds(0, CHUNK)]
        pltpu.make_async_copy(z_sh.at[idx0], srcr[b], sem_s[b]).wait()
        pltpu.make_async_copy(z_sh.at[idx0], dstr[b], sem_d[b]).wait()

    def compute(ci, b):
        @plsc.parallel_loop(0, CHUNK, unroll=8)
        def _edge_loop(k):
            kv = jnp.zeros((L,), jnp.int32) + (ci * CHUNK + k)
            rid_b = plsc.load_gather(rid_v, [kv])
            acc0 = jnp.zeros((L,), jnp.float32)
            acc1 = jnp.zeros((L,), jnp.float32)
            for m in range(D // 32):
                sv = srcr[b][k, pl.ds(m * 32, 32)]
                tv = dstr[b][k, pl.ds(m * 32, 32)]
                pe, po = plsc.unpack(sv * tv,
                                     format=plsc.PackFormat.INTERLEAVED)
                rp = plsc.load_gather(rel_v, [pv[m] + rid_b])
                re, ro = plsc.unpack(plsc.bitcast(rp, jnp.bfloat16),
                                     format=plsc.PackFormat.INTERLEAVED)
                acc0 = acc0 + pe * re
                acc1 = acc1 + po * ro
            accs_v[pl.ds(k * ASTR, L)] = acc0 + acc1

        @plsc.parallel_loop(0, GROUPS, unroll=1)
        def _reduce_loop(g):
            base = g * (L * ASTR)
            parts = []
            for p4 in range(4):
                t = plsc.load_gather(accs_v, [base + iota_astr + 4 * p4])
                for c in range(1, 4):
                    t = t + plsc.load_gather(
                        accs_v, [base + iota_astr + 4 * p4 + c])
                parts.append(t)
            tot = (parts[0] + parts[1]) + (parts[2] + parts[3])
            sig = 1.0 / (1.0 + jnp.exp(-tot))
            out_v[pl.ds(ci * CHUNK + g * L, L)] = sig

    issue(0, 0)

    def body(k, carry):
        ci0 = 2 * k
        ci1 = 2 * k + 1
        issue(ci1, 1)
        drain(0)
        compute(ci0, 0)

        @pl.when(ci0 + 2 < N_CHUNKS)
        def _():
            issue(ci0 + 2, 0)

        drain(1)
        compute(ci1, 1)
        return carry

    lax.fori_loop(0, N_CHUNKS // 2, body, 0)
    drain(0)
    compute(N_CHUNKS - 1, 0)
    pltpu.sync_copy(out_v, out_hbm.at[pl.ds(base_w, PER_W)])


def kernel(z, edge_index, rel_id, rel):
    src = edge_index[0].astype(jnp.int32)
    dst = edge_index[1].astype(jnp.int32)
    rid = rel_id.astype(jnp.int32)
    z_bf = z.astype(jnp.bfloat16)
    q = np.arange(D // 2)
    f_even = 32 * (q // 16) + 2 * (q % 16)
    rt = rel.astype(jnp.float32).T
    e16 = jax.lax.bitcast_convert_type(
        rt[f_even].astype(jnp.bfloat16), jnp.uint16).astype(jnp.uint32)
    o16 = jax.lax.bitcast_convert_type(
        rt[f_even + 1].astype(jnp.bfloat16), jnp.uint16).astype(jnp.uint32)
    packed = jax.lax.bitcast_convert_type(e16 | (o16 << 16), jnp.float32)
    rel_t = jnp.pad(packed, ((0, 0), (0, RSTR - REL_TYPES))).reshape(-1)
    return _distmult_sc(z_bf, src, dst, rid, rel_t)


# speedup vs baseline: 15.4892x; 1.0068x over previous
import functools

import jax
import jax.numpy as jnp
import numpy as np
from jax import lax
from jax.experimental import pallas as pl
from jax.experimental.pallas import tpu as pltpu, tpu_sc as plsc

N_NODES = 10000
N_EDGES = 320000
D = 128
REL_TYPES = 16
RSTR = REL_TYPES + 1

_info = plsc.get_sparse_core_info()
NC, NS, L = _info.num_cores, _info.num_subcores, _info.num_lanes
NW = NC * NS
PER_W = N_EDGES // NW
SUB = 80
CHUNK = 2 * SUB
N_FULL = PER_W // CHUNK
TAIL = PER_W - N_FULL * CHUNK
ASTR = L + 1


@functools.partial(
    pl.kernel,
    mesh=plsc.VectorSubcoreMesh(core_axis_name="c", subcore_axis_name="s"),
    out_type=jax.ShapeDtypeStruct((N_EDGES,), jnp.float32),
    scratch_types=[
        pltpu.VMEM((PER_W,), jnp.int32),
        pltpu.VMEM((PER_W,), jnp.int32),
        pltpu.VMEM((PER_W,), jnp.int32),
        pltpu.VMEM((CHUNK, D), jnp.bfloat16),
        pltpu.VMEM((CHUNK, D), jnp.bfloat16),
        pltpu.VMEM((CHUNK, D), jnp.bfloat16),
        pltpu.VMEM((CHUNK, D), jnp.bfloat16),
        pltpu.VMEM((D // 2 * RSTR,), jnp.float32),
        pltpu.VMEM((CHUNK * ASTR,), jnp.float32),
        pltpu.VMEM((PER_W,), jnp.float32),
        pltpu.VMEM_SHARED((N_NODES, D), jnp.bfloat16),
        pltpu.SemaphoreType.DMA,
        pltpu.SemaphoreType.DMA,
        pltpu.SemaphoreType.DMA,
        pltpu.SemaphoreType.DMA,
    ],
    compiler_params=pltpu.CompilerParams(needs_layout_passes=False,
                                         use_tc_tiling_on_sc=False),
)
def _distmult_sc(z_hbm, src_hbm, dst_hbm, rid_hbm, rel_hbm, out_hbm,
                 srci_v, dsti_v, rid_v, srcr0, srcr1, dstr0, dstr1,
                 rel_v, accs_v, out_v, z_sh, sem_s0, sem_s1, sem_d0, sem_d1):
    wid = lax.axis_index("s") * NC + lax.axis_index("c")
    base_w = wid * PER_W
    sid = lax.axis_index("s")
    zrows = N_NODES // NS
    pltpu.sync_copy(z_hbm.at[pl.ds(sid * zrows, zrows)],
                    z_sh.at[pl.ds(sid * zrows, zrows)])
    pltpu.sync_copy(rel_hbm, rel_v)
    pltpu.sync_copy(src_hbm.at[pl.ds(base_w, PER_W)], srci_v)
    pltpu.sync_copy(dst_hbm.at[pl.ds(base_w, PER_W)], dsti_v)
    pltpu.sync_copy(rid_hbm.at[pl.ds(base_w, PER_W)], rid_v)
    plsc.subcore_barrier()
    iota16 = lax.iota(jnp.int32, L)
    iota_astr = iota16 * ASTR
    pv = [(16 * m + iota16) * RSTR for m in range(D // 32)]

    srcr = (srcr0, srcr1)
    dstr = (dstr0, dstr1)
    sem_s = (sem_s0, sem_s1)
    sem_d = (sem_d0, sem_d1)

    def issue(base, b, nh):
        for h in range(nh):
            idx_s = srci_v.at[pl.ds(base + SUB * h, SUB)]
            idx_d = dsti_v.at[pl.ds(base + SUB * h, SUB)]
            dst_s = srcr[b].at[pl.ds(SUB * h, SUB)]
            dst_d = dstr[b].at[pl.ds(SUB * h, SUB)]
            pltpu.async_copy(z_sh.at[idx_s], dst_s, sem_s[b])
            pltpu.async_copy(z_sh.at[idx_d], dst_d, sem_d[b])

    def drain(b, nh):
        idx0 = srci_v.at[pl.ds(0, SUB)]
        for h in range(nh):
            pltpu.make_async_copy(
                z_sh.at[idx0], srcr[b].at[pl.ds(SUB * h, SUB)],
                sem_s[b]).wait()
            pltpu.make_async_copy(
                z_sh.at[idx0], dstr[b].at[pl.ds(SUB * h, SUB)],
                sem_d[b]).wait()

    def compute(base, b, n):
        @plsc.parallel_loop(0, n, unroll=8)
        def _edge_loop(k):
            kv = jnp.zeros((L,), jnp.int32) + (base + k)
            rid_b = plsc.load_gather(rid_v, [kv])
            acc0 = jnp.zeros((L,), jnp.float32)
            acc1 = jnp.zeros((L,), jnp.float32)
            for m in range(D // 32):
                sv = srcr[b][k, pl.ds(m * 32, 32)]
                tv = dstr[b][k, pl.ds(m * 32, 32)]
                pe, po = plsc.unpack(sv * tv,
                                     format=plsc.PackFormat.INTERLEAVED)
                rp = plsc.load_gather(rel_v, [pv[m] + rid_b])
                re, ro = plsc.unpack(plsc.bitcast(rp, jnp.bfloat16),
                                     format=plsc.PackFormat.INTERLEAVED)
                acc0 = acc0 + pe * re
                acc1 = acc1 + po * ro
            accs_v[pl.ds(k * ASTR, L)] = acc0 + acc1

        @plsc.parallel_loop(0, n // L, unroll=1)
        def _reduce_loop(g):
            gb = g * (L * ASTR)
            parts = []
            for p4 in range(4):
                t = plsc.load_gather(accs_v, [gb + iota_astr + 4 * p4])
                for c in range(1, 4):
                    t = t + plsc.load_gather(
                        accs_v, [gb + iota_astr + 4 * p4 + c])
                parts.append(t)
            tot = (parts[0] + parts[1]) + (parts[2] + parts[3])
            sig = 1.0 / (1.0 + jnp.exp(-tot))
            out_v[pl.ds(base + g * L, L)] = sig

    issue(0, 0, 2)

    def body(kk, carry):
        c0 = kk * (2 * CHUNK)
        c1 = c0 + CHUNK
        issue(c1, 1, 2)
        drain(0, 2)
        compute(c0, 0, CHUNK)

        @pl.when(c0 + 2 * CHUNK < N_FULL * CHUNK)
        def _():
            issue(c0 + 2 * CHUNK, 0, 2)

        drain(1, 2)
        compute(c1, 1, CHUNK)
        return carry

    lax.fori_loop(0, N_FULL // 2, body, 0)
    issue(N_FULL * CHUNK, 0, 1)
    drain(0, 1)
    compute(N_FULL * CHUNK, 0, TAIL)
    pltpu.sync_copy(out_v, out_hbm.at[pl.ds(base_w, PER_W)])


def kernel(z, edge_index, rel_id, rel):
    src = edge_index[0].astype(jnp.int32)
    dst = edge_index[1].astype(jnp.int32)
    rid = rel_id.astype(jnp.int32)
    z_bf = z.astype(jnp.bfloat16)
    q = np.arange(D // 2)
    f_even = 32 * (q // 16) + 2 * (q % 16)
    rt = rel.astype(jnp.float32).T
    e16 = jax.lax.bitcast_convert_type(
        rt[f_even].astype(jnp.bfloat16), jnp.uint16).astype(jnp.uint32)
    o16 = jax.lax.bitcast_convert_type(
        rt[f_even + 1].astype(jnp.bfloat16), jnp.uint16).astype(jnp.uint32)
    packed = jax.lax.bitcast_convert_type(e16 | (o16 << 16), jnp.float32)
    rel_t = jnp.pad(packed, ((0, 0), (0, RSTR - REL_TYPES))).reshape(-1)
    return _distmult_sc(z_bf, src, dst, rid, rel_t)
